# baseline (speedup 1.0000x reference)
import functools

import jax
import jax.numpy as jnp
from jax import lax
from jax.experimental import pallas as pl
from jax.experimental.pallas import tpu as pltpu
from jax.experimental.pallas import tpu_sc as plsc

N = 10000
E = 320000
D = 128
DH = D // 2
G = 64
NC = 2
NS = 16
NW = NC * NS
K = 80
CPT = E // (NS * K)
CPW = E // (NW * K)
RPT = 624
TAIL = N - NS * RPT
ZROWS = 208

_mesh = plsc.VectorSubcoreMesh(core_axis_name="c", subcore_axis_name="s")
_sc_params = pltpu.CompilerParams(use_tc_tiling_on_sc=False)


@functools.partial(
    pl.kernel,
    out_type=jax.ShapeDtypeStruct((NC, N), jnp.float32),
    mesh=_mesh,
    scratch_types=[
        pltpu.VMEM((CPW, K), jnp.int32),
        pltpu.VMEM((K,), jnp.float32),
        pltpu.VMEM((N,), jnp.float32),
        pltpu.VMEM_SHARED((N,), jnp.float32),
    ],
    compiler_params=_sc_params,
)
def _sc_degree(dst_hbm, out_hbm, dstv, ones_v, zbuf, deg_sh):
    c = lax.axis_index("c")
    s = lax.axis_index("s")
    w = c * NS + s

    for i in range(K // 16):
        ones_v[pl.ds(i * 16, 16)] = jnp.ones((16,), jnp.float32)

    @pl.when(s == 0)
    def _zero():
        def zb(i, _):
            zbuf[pl.ds(i * 16, 16)] = jnp.zeros((16,), jnp.float32)
            return 0
        lax.fori_loop(0, N // 16, zb, 0)
        pltpu.sync_copy(zbuf, deg_sh)

    pltpu.sync_copy(dst_hbm.at[w], dstv)
    plsc.subcore_barrier()

    def body(j, _):
        pltpu.sync_copy(ones_v, deg_sh.at[dstv.at[j]], add=True)
        return 0
    lax.fori_loop(0, CPW, body, 0)

    plsc.subcore_barrier()

    @pl.when(s == 0)
    def _writeout():
        pltpu.sync_copy(deg_sh, out_hbm.at[c])


@functools.partial(
    pl.kernel,
    out_type=jax.ShapeDtypeStruct((NC, N, DH), jnp.float32),
    mesh=_mesh,
    scratch_types=[
        pltpu.VMEM((CPT, K), jnp.int32),
        pltpu.VMEM((CPT, K), jnp.int32),
        [pltpu.VMEM((K, DH), jnp.float32)] * 5,
        pltpu.VMEM((ZROWS, DH), jnp.float32),
        pltpu.VMEM_SHARED((N, DH), jnp.float32),
        [pltpu.SemaphoreType.DMA] * 5,
    ],
    compiler_params=_sc_params,
)
def _sc_scatter(g_hbm, src_hbm, dst_hbm, out_hbm, srcv, dstv, rows,
                zbuf, acc_sh, sems):
    c = lax.axis_index("c")
    s = lax.axis_index("s")

    def zb(i, _):
        for jj in range(DH // 16):
            zbuf[i, pl.ds(jj * 16, 16)] = jnp.zeros((16,), jnp.float32)
        return 0
    lax.fori_loop(0, ZROWS, zb, 0)
    for kk in range(RPT // ZROWS):
        pltpu.sync_copy(zbuf, acc_sh.at[pl.ds(s * RPT + kk * ZROWS, ZROWS)])

    @pl.when(s == 0)
    def _zero_tail():
        pltpu.sync_copy(zbuf.at[pl.ds(0, TAIL)], acc_sh.at[pl.ds(NS * RPT, TAIL)])

    pltpu.sync_copy(src_hbm.at[s], srcv)
    pltpu.sync_copy(dst_hbm.at[s], dstv)
    plsc.subcore_barrier()

    gc = g_hbm.at[c]

    NBUF = 5
    for b in range(NBUF):
        pltpu.async_copy(gc.at[srcv.at[b]], rows[b], sems[b])

    def body(t, _):
        for b in range(NBUF):
            j = NBUF * t + b
            pltpu.make_async_copy(gc.at[srcv.at[j]], rows[b], sems[b]).wait()
            pltpu.sync_copy(rows[b], acc_sh.at[dstv.at[j]], add=True)

            @pl.when(j + NBUF < CPT)
            def _g():
                pltpu.async_copy(gc.at[srcv.at[j + NBUF]], rows[b], sems[b])
        return 0
    lax.fori_loop(0, CPT // NBUF, body, 0)

    plsc.subcore_barrier()
    pltpu.sync_copy(acc_sh.at[pl.ds(s * RPT, RPT)], out_hbm.at[c, pl.ds(s * RPT, RPT)])

    @pl.when(s == 0)
    def _write_tail():
        pltpu.sync_copy(acc_sh.at[pl.ds(NS * RPT, TAIL)],
                        out_hbm.at[c, pl.ds(NS * RPT, TAIL)])


_HI = lax.Precision.HIGHEST
BN = 10000
NB = N // BN

_b_acc = pl.BlockSpec((NC, BN, DH), lambda i: (0, i, 0))
_b_rows = pl.BlockSpec((BN, D), lambda i: (i, 0))
_b_dis = pl.BlockSpec((BN, 1), lambda i: (i, 0))
_b_row128 = pl.BlockSpec((1, D), lambda i: (0, 0))
_b_full = lambda shape: pl.BlockSpec(shape, lambda i: tuple(0 for _ in shape))


def _tc_first_body(x_ref, w_ref, degt_ref, g_ref, dis_ref):
    deg = jnp.sum(degt_ref[...], axis=1, keepdims=True) + 1.0
    dis = lax.rsqrt(deg)
    g = jnp.dot(x_ref[...], w_ref[...],
                preferred_element_type=jnp.float32) * dis
    g_ref[0] = g[:, :DH]
    g_ref[1] = g[:, DH:]
    dis_ref[...] = dis


def _tc_first(x, W1, degT):
    return pl.pallas_call(
        _tc_first_body,
        grid=(NB,),
        in_specs=[_b_rows, _b_full((D, D)), pl.BlockSpec((BN, 2), lambda i: (i, 0))],
        out_specs=[_b_acc, _b_dis],
        out_shape=[
            jax.ShapeDtypeStruct((NC, N, DH), jnp.float32),
            jax.ShapeDtypeStruct((N, 1), jnp.float32),
        ],
    )(x, W1, degT)


def _tc_block_a(acc, g, dis, b):
    def body(acc_ref, g_ref, dis_ref, b_ref, h_ref, s_ref, q_ref):
        dis = dis_ref[...]
        h = jnp.concatenate(
            [acc_ref[0] + g_ref[0], acc_ref[1] + g_ref[1]], axis=1)
        h = jnp.maximum(dis * h + b_ref[...], 0.0)
        h_ref[...] = h
        s_ref[...] = jnp.sum(h, axis=0, keepdims=True)[None]
        q_ref[...] = jnp.sum(h * h, axis=0, keepdims=True)[None]
    return pl.pallas_call(
        body,
        grid=(NB,),
        in_specs=[_b_acc, _b_acc, _b_dis, _b_row128],
        out_specs=[_b_rows,
                   pl.BlockSpec((1, 1, D), lambda i: (i, 0, 0)),
                   pl.BlockSpec((1, 1, D), lambda i: (i, 0, 0))],
        out_shape=[
            jax.ShapeDtypeStruct((N, D), jnp.float32),
            jax.ShapeDtypeStruct((NB, 1, D), jnp.float32),
            jax.ShapeDtypeStruct((NB, 1, D), jnp.float32),
        ],
    )(acc, g, dis, b)


def _tc_block_b(h, s, q, dis, gamma, beta, Wn, flags):
    def body(h_ref, s_ref, q_ref, dis_ref, gamma_ref, beta_ref, w_ref,
             flags_ref, out_ref):
        f_bn = flags_ref[0, 0] > 0.0
        f_last = flags_ref[0, 1] > 0.0
        mean = jnp.sum(s_ref[...], axis=0) * (1.0 / N)
        var = jnp.sum(q_ref[...], axis=0) * (1.0 / N) - mean * mean
        h = h_ref[...]
        hbn = (gamma_ref[...] * (h - mean) * lax.rsqrt(var + 1e-5)
               + beta_ref[...])
        hsel = jnp.where(f_bn, hbn, h)
        gn = jnp.dot(hsel, w_ref[...],
                     preferred_element_type=jnp.float32) * dis_ref[...]
        res = jnp.where(f_last, hsel, gn)
        out_ref[0] = res[:, :DH]
        out_ref[1] = res[:, DH:]
    return pl.pallas_call(
        body,
        grid=(NB,),
        in_specs=[_b_rows, _b_full((NB, 1, D)), _b_full((NB, 1, D)), _b_dis,
                  _b_row128, _b_row128, _b_full((D, D)), _b_full((1, 2))],
        out_specs=_b_acc,
        out_shape=jax.ShapeDtypeStruct((NC, N, DH), jnp.float32),
    )(h, s, q, dis, gamma, beta, Wn, flags)


def _tc_pool_body(h_ref, batch_ref, fcw_ref, fcb_ref, out_ref, ssum, cnt):
    i = pl.program_id(0)

    @pl.when(i == 0)
    def _init():
        ssum[...] = jnp.zeros((G, D), jnp.float32)
        cnt[...] = jnp.zeros((G, 128), jnp.float32)

    h = jnp.concatenate([h_ref[0], h_ref[1]], axis=1)
    gids = lax.broadcasted_iota(jnp.int32, (BN, G), 1)
    oht = (gids == batch_ref[...]).astype(jnp.float32)
    dn = (((0,), (0,)), ((), ()))
    ssum[...] += lax.dot_general(oht, h, dn, precision=_HI,
                                 preferred_element_type=jnp.float32)
    cnt[...] += lax.dot_general(oht, jnp.ones((BN, 1), jnp.float32), dn,
                                precision=_HI,
                                preferred_element_type=jnp.float32)

    @pl.when(i == NB - 1)
    def _fc():
        pooled = ssum[...] / jnp.maximum(cnt[:, :1], 1.0)
        out_ref[...] = jnp.dot(pooled, fcw_ref[...],
                               preferred_element_type=jnp.float32) + fcb_ref[...]


def _tc_pool(h, batch_row, fc_W, fc_b):
    return pl.pallas_call(
        _tc_pool_body,
        grid=(NB,),
        in_specs=[_b_acc, pl.BlockSpec((BN, 1), lambda i: (i, 0)),
                  _b_full((D, 10)), _b_full((1, 10))],
        out_specs=_b_full((G, 10)),
        out_shape=jax.ShapeDtypeStruct((G, 10), jnp.float32),
        scratch_shapes=[pltpu.VMEM((G, D), jnp.float32),
                        pltpu.VMEM((G, 128), jnp.float32)],
    )(h, batch_row, fc_W, fc_b)


def kernel(x, edge_index, batch, W1, b1, W2, b2, W3, b3,
           gamma1, beta1, gamma2, beta2, fc_W, fc_b):
    src_t = edge_index[0].astype(jnp.int32).reshape(NS, CPT, K)
    dst_t = edge_index[1].astype(jnp.int32).reshape(NS, CPT, K)
    dst_w = edge_index[1].astype(jnp.int32).reshape(NW, CPW, K)
    batch_col = batch.astype(jnp.int32).reshape(N, 1)

    deg_parts = _sc_degree(dst_w)
    degT = jnp.transpose(deg_parts)

    g1, dis = _tc_first(x, W1, degT)

    ones_r = jnp.ones((1, D), jnp.float32)
    zeros_r = jnp.zeros((1, D), jnp.float32)
    Wn_stack = jnp.stack([W2, W3, W3])
    b_stack = jnp.stack([b1.reshape(1, D), b2.reshape(1, D), b3.reshape(1, D)])
    gamma_stack = jnp.stack([ones_r, gamma1.reshape(1, D), gamma2.reshape(1, D)])
    beta_stack = jnp.stack([zeros_r, beta1.reshape(1, D), beta2.reshape(1, D)])
    flag_stack = jnp.array([[[0.0, 0.0]], [[1.0, 0.0]], [[1.0, 1.0]]],
                           jnp.float32)

    def body(l, g):
        acc = _sc_scatter(g, src_t, dst_t)
        Wl = lax.dynamic_index_in_dim(Wn_stack, l, 0, keepdims=False)
        bl = lax.dynamic_index_in_dim(b_stack, l, 0, keepdims=False)
        gml = lax.dynamic_index_in_dim(gamma_stack, l, 0, keepdims=False)
        btl = lax.dynamic_index_in_dim(beta_stack, l, 0, keepdims=False)
        fl = lax.dynamic_index_in_dim(flag_stack, l, 0, keepdims=False)
        h, s, q = _tc_block_a(acc, g, dis, bl)
        return _tc_block_b(h, s, q, dis, gml, btl, Wl, fl)

    h3 = lax.fori_loop(0, 3, body, g1)
    return _tc_pool(h3, batch_col, fc_W, fc_b.reshape(1, 10))


# speedup vs baseline: 1.0261x; 1.0261x over previous
import functools

import jax
import jax.numpy as jnp
from jax import lax
from jax.experimental import pallas as pl
from jax.experimental.pallas import tpu as pltpu
from jax.experimental.pallas import tpu_sc as plsc

N = 10000
E = 320000
D = 128
DH = D // 2
G = 64
NC = 2
NS = 16
NW = NC * NS
K = 80
CPT = E // (NS * K)
CPW = E // (NW * K)
RPT = 624
TAIL = N - NS * RPT
ZROWS = 208

_mesh = plsc.VectorSubcoreMesh(core_axis_name="c", subcore_axis_name="s")
_sc_params = pltpu.CompilerParams(use_tc_tiling_on_sc=False)


@functools.partial(
    pl.kernel,
    out_type=jax.ShapeDtypeStruct((NC, N), jnp.float32),
    mesh=_mesh,
    scratch_types=[
        pltpu.VMEM((CPW, K), jnp.int32),
        pltpu.VMEM((K,), jnp.float32),
        pltpu.VMEM((N,), jnp.float32),
        pltpu.VMEM_SHARED((N,), jnp.float32),
    ],
    compiler_params=_sc_params,
)
def _sc_degree(dst_hbm, out_hbm, dstv, ones_v, zbuf, deg_sh):
    c = lax.axis_index("c")
    s = lax.axis_index("s")
    w = c * NS + s

    for i in range(K // 16):
        ones_v[pl.ds(i * 16, 16)] = jnp.ones((16,), jnp.float32)

    @pl.when(s == 0)
    def _zero():
        def zb(i, _):
            zbuf[pl.ds(i * 16, 16)] = jnp.zeros((16,), jnp.float32)
            return 0
        lax.fori_loop(0, N // 16, zb, 0)
        pltpu.sync_copy(zbuf, deg_sh)

    pltpu.sync_copy(dst_hbm.at[w], dstv)
    plsc.subcore_barrier()

    def body(j, _):
        pltpu.sync_copy(ones_v, deg_sh.at[dstv.at[j]], add=True)
        return 0
    lax.fori_loop(0, CPW, body, 0)

    plsc.subcore_barrier()

    @pl.when(s == 0)
    def _writeout():
        pltpu.sync_copy(deg_sh, out_hbm.at[c])


@functools.partial(
    pl.kernel,
    out_type=jax.ShapeDtypeStruct((NC, N, DH), jnp.float32),
    mesh=_mesh,
    scratch_types=[
        pltpu.VMEM((CPT, K), jnp.int32),
        pltpu.VMEM((CPT, K), jnp.int32),
        [pltpu.VMEM((K, DH), jnp.float32)] * 5,
        pltpu.VMEM((ZROWS, DH), jnp.float32),
        pltpu.VMEM_SHARED((N, DH), jnp.float32),
        [pltpu.SemaphoreType.DMA] * 5,
    ],
    compiler_params=_sc_params,
)
def _sc_scatter(g_hbm, src_hbm, dst_hbm, out_hbm, srcv, dstv, rows,
                zbuf, acc_sh, sems):
    c = lax.axis_index("c")
    s = lax.axis_index("s")

    def zb(i, _):
        for jj in range(DH // 16):
            zbuf[i, pl.ds(jj * 16, 16)] = jnp.zeros((16,), jnp.float32)
        return 0
    lax.fori_loop(0, ZROWS, zb, 0)
    for kk in range(RPT // ZROWS):
        pltpu.sync_copy(zbuf, acc_sh.at[pl.ds(s * RPT + kk * ZROWS, ZROWS)])

    @pl.when(s == 0)
    def _zero_tail():
        pltpu.sync_copy(zbuf.at[pl.ds(0, TAIL)], acc_sh.at[pl.ds(NS * RPT, TAIL)])

    pltpu.sync_copy(src_hbm.at[s], srcv)
    pltpu.sync_copy(dst_hbm.at[s], dstv)
    plsc.subcore_barrier()

    gc = g_hbm.at[c]

    NBUF = 5
    for b in range(NBUF):
        pltpu.async_copy(gc.at[srcv.at[b]], rows[b], sems[b])

    def body(t, _):
        for b in range(NBUF):
            j = NBUF * t + b
            pltpu.make_async_copy(gc.at[srcv.at[j]], rows[b], sems[b]).wait()
            pltpu.sync_copy(rows[b], acc_sh.at[dstv.at[j]], add=True)

            @pl.when(j + NBUF < CPT)
            def _g():
                pltpu.async_copy(gc.at[srcv.at[j + NBUF]], rows[b], sems[b])
        return 0
    lax.fori_loop(0, CPT // NBUF, body, 0)

    plsc.subcore_barrier()
    pltpu.sync_copy(acc_sh.at[pl.ds(s * RPT, RPT)], out_hbm.at[c, pl.ds(s * RPT, RPT)])

    @pl.when(s == 0)
    def _write_tail():
        pltpu.sync_copy(acc_sh.at[pl.ds(NS * RPT, TAIL)],
                        out_hbm.at[c, pl.ds(NS * RPT, TAIL)])


_HI = lax.Precision.HIGHEST
BN = 5000
NB = N // BN

_b_acc = pl.BlockSpec((NC, BN, DH), lambda i: (0, i, 0))
_b_rows = pl.BlockSpec((BN, D), lambda i: (i, 0))
_b_dis = pl.BlockSpec((BN, 1), lambda i: (i, 0))
_b_row128 = pl.BlockSpec((1, D), lambda i: (0, 0))
_b_full = lambda shape: pl.BlockSpec(shape, lambda i: tuple(0 for _ in shape))


def _tc_first_body(x_ref, w_ref, degt_ref, g_ref, dis_ref):
    deg = jnp.sum(degt_ref[...], axis=1, keepdims=True) + 1.0
    dis = lax.rsqrt(deg)
    g = jnp.dot(x_ref[...], w_ref[...],
                preferred_element_type=jnp.float32) * dis
    g_ref[0] = g[:, :DH]
    g_ref[1] = g[:, DH:]
    dis_ref[...] = dis


def _tc_first(x, W1, degT):
    return pl.pallas_call(
        _tc_first_body,
        grid=(NB,),
        in_specs=[_b_rows, _b_full((D, D)), pl.BlockSpec((BN, 2), lambda i: (i, 0))],
        out_specs=[_b_acc, _b_dis],
        out_shape=[
            jax.ShapeDtypeStruct((NC, N, DH), jnp.float32),
            jax.ShapeDtypeStruct((N, 1), jnp.float32),
        ],
    )(x, W1, degT)


def _tc_block_a(acc, g, dis, b):
    def body(acc_ref, g_ref, dis_ref, b_ref, h_ref, s_ref, q_ref):
        dis = dis_ref[...]
        h = jnp.concatenate(
            [acc_ref[0] + g_ref[0], acc_ref[1] + g_ref[1]], axis=1)
        h = jnp.maximum(dis * h + b_ref[...], 0.0)
        h_ref[...] = h
        s_ref[...] = jnp.sum(h, axis=0, keepdims=True)[None]
        q_ref[...] = jnp.sum(h * h, axis=0, keepdims=True)[None]
    return pl.pallas_call(
        body,
        grid=(NB,),
        in_specs=[_b_acc, _b_acc, _b_dis, _b_row128],
        out_specs=[_b_rows,
                   pl.BlockSpec((1, 1, D), lambda i: (i, 0, 0)),
                   pl.BlockSpec((1, 1, D), lambda i: (i, 0, 0))],
        out_shape=[
            jax.ShapeDtypeStruct((N, D), jnp.float32),
            jax.ShapeDtypeStruct((NB, 1, D), jnp.float32),
            jax.ShapeDtypeStruct((NB, 1, D), jnp.float32),
        ],
    )(acc, g, dis, b)


def _tc_block_b(h, s, q, dis, gamma, beta, Wn, flags):
    def body(h_ref, s_ref, q_ref, dis_ref, gamma_ref, beta_ref, w_ref,
             flags_ref, out_ref):
        f_bn = flags_ref[0, 0] > 0.0
        f_last = flags_ref[0, 1] > 0.0
        mean = jnp.sum(s_ref[...], axis=0) * (1.0 / N)
        var = jnp.sum(q_ref[...], axis=0) * (1.0 / N) - mean * mean
        h = h_ref[...]
        hbn = (gamma_ref[...] * (h - mean) * lax.rsqrt(var + 1e-5)
               + beta_ref[...])
        hsel = jnp.where(f_bn, hbn, h)
        gn = jnp.dot(hsel, w_ref[...],
                     preferred_element_type=jnp.float32) * dis_ref[...]
        res = jnp.where(f_last, hsel, gn)
        out_ref[0] = res[:, :DH]
        out_ref[1] = res[:, DH:]
    return pl.pallas_call(
        body,
        grid=(NB,),
        in_specs=[_b_rows, _b_full((NB, 1, D)), _b_full((NB, 1, D)), _b_dis,
                  _b_row128, _b_row128, _b_full((D, D)), _b_full((1, 2))],
        out_specs=_b_acc,
        out_shape=jax.ShapeDtypeStruct((NC, N, DH), jnp.float32),
    )(h, s, q, dis, gamma, beta, Wn, flags)


def _tc_pool_body(h_ref, batch_ref, fcw_ref, fcb_ref, out_ref, ssum, cnt):
    i = pl.program_id(0)

    @pl.when(i == 0)
    def _init():
        ssum[...] = jnp.zeros((G, D), jnp.float32)
        cnt[...] = jnp.zeros((G, 128), jnp.float32)

    h = jnp.concatenate([h_ref[0], h_ref[1]], axis=1)
    gids = lax.broadcasted_iota(jnp.int32, (BN, G), 1)
    oht = (gids == batch_ref[...]).astype(jnp.float32)
    dn = (((0,), (0,)), ((), ()))
    ssum[...] += lax.dot_general(oht, h, dn, precision=_HI,
                                 preferred_element_type=jnp.float32)
    cnt[...] += lax.dot_general(oht, jnp.ones((BN, 1), jnp.float32), dn,
                                precision=_HI,
                                preferred_element_type=jnp.float32)

    @pl.when(i == NB - 1)
    def _fc():
        pooled = ssum[...] / jnp.maximum(cnt[:, :1], 1.0)
        out_ref[...] = jnp.dot(pooled, fcw_ref[...],
                               preferred_element_type=jnp.float32) + fcb_ref[...]


def _tc_pool(h, batch_row, fc_W, fc_b):
    return pl.pallas_call(
        _tc_pool_body,
        grid=(NB,),
        in_specs=[_b_acc, pl.BlockSpec((BN, 1), lambda i: (i, 0)),
                  _b_full((D, 10)), _b_full((1, 10))],
        out_specs=_b_full((G, 10)),
        out_shape=jax.ShapeDtypeStruct((G, 10), jnp.float32),
        scratch_shapes=[pltpu.VMEM((G, D), jnp.float32),
                        pltpu.VMEM((G, 128), jnp.float32)],
    )(h, batch_row, fc_W, fc_b)


def kernel(x, edge_index, batch, W1, b1, W2, b2, W3, b3,
           gamma1, beta1, gamma2, beta2, fc_W, fc_b):
    src_t = edge_index[0].astype(jnp.int32).reshape(NS, CPT, K)
    dst_t = edge_index[1].astype(jnp.int32).reshape(NS, CPT, K)
    dst_w = edge_index[1].astype(jnp.int32).reshape(NW, CPW, K)
    batch_col = batch.astype(jnp.int32).reshape(N, 1)

    deg_parts = _sc_degree(dst_w)
    degT = jnp.transpose(deg_parts)

    g1, dis = _tc_first(x, W1, degT)

    ones_r = jnp.ones((1, D), jnp.float32)
    zeros_r = jnp.zeros((1, D), jnp.float32)
    Wn_stack = jnp.stack([W2, W3, W3])
    b_stack = jnp.stack([b1.reshape(1, D), b2.reshape(1, D), b3.reshape(1, D)])
    gamma_stack = jnp.stack([ones_r, gamma1.reshape(1, D), gamma2.reshape(1, D)])
    beta_stack = jnp.stack([zeros_r, beta1.reshape(1, D), beta2.reshape(1, D)])
    flag_stack = jnp.array([[[0.0, 0.0]], [[1.0, 0.0]], [[1.0, 1.0]]],
                           jnp.float32)

    def body(l, g):
        acc = _sc_scatter(g, src_t, dst_t)
        Wl = lax.dynamic_index_in_dim(Wn_stack, l, 0, keepdims=False)
        bl = lax.dynamic_index_in_dim(b_stack, l, 0, keepdims=False)
        gml = lax.dynamic_index_in_dim(gamma_stack, l, 0, keepdims=False)
        btl = lax.dynamic_index_in_dim(beta_stack, l, 0, keepdims=False)
        fl = lax.dynamic_index_in_dim(flag_stack, l, 0, keepdims=False)
        h, s, q = _tc_block_a(acc, g, dis, bl)
        return _tc_block_b(h, s, q, dis, gml, btl, Wl, fl)

    h3 = lax.fori_loop(0, 3, body, g1)
    return _tc_pool(h3, batch_col, fc_W, fc_b.reshape(1, 10))
